# baseline (speedup 1.0000x reference)
import functools

import jax
import jax.numpy as jnp
from jax import lax
from jax.experimental import pallas as pl
from jax.experimental.pallas import tpu as pltpu
from jax.experimental.pallas import tpu_sc as plsc

B = 4096
L = 200
V = 100000
D = 64
NC = 2
NS = 16
NW = NC * NS
SEQ_PER_W = B // NW
NLANE = D // 16
U = 8
SENT = -1


def _pool_body(x_hbm, vl_hbm, tab_hbm, s_hbm, idx_v, vl_v, rows_v, out_v,
               sem0, sem1):
    wid = lax.axis_index("s") * NC + lax.axis_index("c")
    base = wid * SEQ_PER_W

    pltpu.sync_copy(x_hbm.at[pl.ds(base * L, SEQ_PER_W * L)], idx_v)
    pltpu.sync_copy(vl_hbm.at[pl.ds(base, SEQ_PER_W)],
                    vl_v.at[pl.ds(0, SEQ_PER_W)])

    sems = (sem0, sem1)
    iota = lax.iota(jnp.int32, 16)

    def mask_and_issue(i, buf):
        vl = vl_v[pl.ds(i, 16)][0]
        vlv = jnp.full((16,), vl, jnp.int32)
        for c in range(L // 16 + 1):
            o = min(c * 16, L - 16)
            cur = idx_v[pl.ds(i * L + o, 16)]
            idx_v[pl.ds(i * L + o, 16)] = jnp.where(iota + o < vlv, cur, SENT)
        pltpu.async_copy(
            tab_hbm.at[plsc.Indices(idx_v.at[pl.ds(i * L, L)],
                                    ignored_value=SENT)],
            rows_v.at[buf], sems[buf])

    def drain(buf):
        pltpu.make_async_copy(
            tab_hbm.at[plsc.Indices(idx_v.at[pl.ds(0, L)],
                                    ignored_value=SENT)],
            rows_v.at[buf], sems[buf]).wait()

    def accum(i, buf):
        vl = vl_v[pl.ds(i, 16)][0]
        zero = jnp.zeros((16,), jnp.float32)
        init = tuple(zero for _ in range(2 * NLANE))

        def main_body(t, accs):
            j0 = t * U
            a = list(accs)
            for u in range(U):
                half = (u % 2) * NLANE
                for d in range(NLANE):
                    a[half + d] = a[half + d] + rows_v[buf, j0 + u,
                                                       pl.ds(d * 16, 16)]
            return tuple(a)

        nfull = vl // U
        accs = lax.fori_loop(0, nfull, main_body, init)

        def tail_body(j, accs):
            a = list(accs)
            for d in range(NLANE):
                a[d] = a[d] + rows_v[buf, j, pl.ds(d * 16, 16)]
            return tuple(a)

        accs = lax.fori_loop(nfull * U, vl, tail_body, accs)
        for d in range(NLANE):
            out_v[pl.ds(i * D + d * 16, 16)] = accs[d] + accs[NLANE + d]

    mask_and_issue(0, 0)

    @pl.loop(0, SEQ_PER_W // 2)
    def _(t):
        i0 = 2 * t
        mask_and_issue(i0 + 1, 1)
        drain(0)
        accum(i0, 0)

        @pl.when(t < SEQ_PER_W // 2 - 1)
        def _():
            mask_and_issue(i0 + 2, 0)

        drain(1)
        accum(i0 + 1, 1)

    pltpu.sync_copy(out_v, s_hbm.at[pl.ds(base * D, SEQ_PER_W * D)])


_pool = functools.partial(
    pl.kernel,
    _pool_body,
    out_type=jax.ShapeDtypeStruct((B * D,), jnp.float32),
    mesh=plsc.VectorSubcoreMesh(core_axis_name="c", subcore_axis_name="s",
                                num_cores=NC, num_subcores=NS),
    compiler_params=pltpu.CompilerParams(use_tc_tiling_on_sc=False),
    scratch_types=[
        pltpu.VMEM((SEQ_PER_W * L,), jnp.int32),
        pltpu.VMEM((SEQ_PER_W + 16,), jnp.int32),
        pltpu.VMEM((2, L, D), jnp.float32),
        pltpu.VMEM((SEQ_PER_W * D,), jnp.float32),
        pltpu.SemaphoreType.DMA,
        pltpu.SemaphoreType.DMA,
    ],
)


def _mlp_body(s_ref, w1_ref, b1_ref, w2_ref, b2_ref, o_ref):
    s = s_ref[...]
    h = jnp.maximum(
        lax.dot(s, w1_ref[...], preferred_element_type=jnp.float32)
        + b1_ref[...], 0.0)
    o_ref[...] = jnp.maximum(
        lax.dot(h, w2_ref[...], preferred_element_type=jnp.float32)
        + b2_ref[...], 0.0)


def kernel(X, valid_lens, table, W1, b1, W2, b2):
    s = _pool()(X.reshape(B * L).astype(jnp.int32),
                valid_lens.astype(jnp.int32), table)
    out = pl.pallas_call(
        _mlp_body,
        out_shape=jax.ShapeDtypeStruct((B, D), jnp.float32),
    )(s.reshape(B, D), W1, b1.reshape(1, D), W2, b2.reshape(1, D))
    return out


# speedup vs baseline: 1.1002x; 1.1002x over previous
import functools

import jax
import jax.numpy as jnp
from jax import lax
from jax.experimental import pallas as pl
from jax.experimental.pallas import tpu as pltpu
from jax.experimental.pallas import tpu_sc as plsc

B = 4096
L = 200
V = 100000
D = 64
NC = 2
NS = 16
NW = NC * NS
SEQ_PER_W = B // NW
NLANE = D // 16
U = 8
SENT = -1


def _pool_body(x_hbm, vl_hbm, tab_hbm, s_hbm, idx_v, vl_v, rows_v, out_v,
               sem0, sem1):
    wid = lax.axis_index("s") * NC + lax.axis_index("c")
    base = wid * SEQ_PER_W

    pltpu.sync_copy(x_hbm.at[pl.ds(base * L, SEQ_PER_W * L)], idx_v)
    pltpu.sync_copy(vl_hbm.at[pl.ds(base, SEQ_PER_W)],
                    vl_v.at[pl.ds(0, SEQ_PER_W)])

    sems = (sem0, sem1)
    iota = lax.iota(jnp.int32, 16)

    def mask_and_issue(p, buf):
        for k in range(2):
            i = 2 * p + k
            vl = vl_v[pl.ds(i, 16)][0]
            vlv = jnp.full((16,), vl, jnp.int32)
            for c in range(L // 16 + 1):
                o = min(c * 16, L - 16)
                cur = idx_v[pl.ds(i * L + o, 16)]
                idx_v[pl.ds(i * L + o, 16)] = jnp.where(iota + o < vlv,
                                                        cur, SENT)
        pltpu.async_copy(
            tab_hbm.at[plsc.Indices(idx_v.at[pl.ds(p * 2 * L, 2 * L)],
                                    ignored_value=SENT)],
            rows_v.at[buf], sems[buf])

    def drain(buf):
        pltpu.make_async_copy(
            tab_hbm.at[plsc.Indices(idx_v.at[pl.ds(0, 2 * L)],
                                    ignored_value=SENT)],
            rows_v.at[buf], sems[buf]).wait()

    def accum(i, buf, roff):
        vl = vl_v[pl.ds(i, 16)][0]
        zero = jnp.zeros((16,), jnp.float32)
        init = tuple(zero for _ in range(2 * NLANE))

        def main_body(t, accs):
            j0 = t * U
            a = list(accs)
            for u in range(U):
                half = (u % 2) * NLANE
                for d in range(NLANE):
                    a[half + d] = a[half + d] + rows_v[buf, roff + j0 + u,
                                                       pl.ds(d * 16, 16)]
            return tuple(a)

        nfull = vl // U
        accs = lax.fori_loop(0, nfull, main_body, init)

        def tail_body(j, accs):
            a = list(accs)
            for d in range(NLANE):
                a[d] = a[d] + rows_v[buf, roff + j, pl.ds(d * 16, 16)]
            return tuple(a)

        accs = lax.fori_loop(nfull * U, vl, tail_body, accs)
        for d in range(NLANE):
            out_v[pl.ds(i * D + d * 16, 16)] = accs[d] + accs[NLANE + d]

    NPAIR = SEQ_PER_W // 2
    mask_and_issue(0, 0)

    @pl.loop(0, NPAIR // 2)
    def _(t):
        p0 = 2 * t
        mask_and_issue(p0 + 1, 1)
        drain(0)
        accum(2 * p0, 0, 0)
        accum(2 * p0 + 1, 0, L)

        @pl.when(t < NPAIR // 2 - 1)
        def _():
            mask_and_issue(p0 + 2, 0)

        drain(1)
        accum(2 * p0 + 2, 1, 0)
        accum(2 * p0 + 3, 1, L)

    pltpu.sync_copy(out_v, s_hbm.at[pl.ds(base * D, SEQ_PER_W * D)])


_pool = functools.partial(
    pl.kernel,
    _pool_body,
    out_type=jax.ShapeDtypeStruct((B * D,), jnp.float32),
    mesh=plsc.VectorSubcoreMesh(core_axis_name="c", subcore_axis_name="s",
                                num_cores=NC, num_subcores=NS),
    compiler_params=pltpu.CompilerParams(use_tc_tiling_on_sc=False),
    scratch_types=[
        pltpu.VMEM((SEQ_PER_W * L,), jnp.int32),
        pltpu.VMEM((SEQ_PER_W + 16,), jnp.int32),
        pltpu.VMEM((2, 2 * L, D), jnp.float32),
        pltpu.VMEM((SEQ_PER_W * D,), jnp.float32),
        pltpu.SemaphoreType.DMA,
        pltpu.SemaphoreType.DMA,
    ],
)


def _mlp_body(s_ref, w1_ref, b1_ref, w2_ref, b2_ref, o_ref):
    s = s_ref[...]
    h = jnp.maximum(
        lax.dot(s, w1_ref[...], preferred_element_type=jnp.float32)
        + b1_ref[...], 0.0)
    o_ref[...] = jnp.maximum(
        lax.dot(h, w2_ref[...], preferred_element_type=jnp.float32)
        + b2_ref[...], 0.0)


def kernel(X, valid_lens, table, W1, b1, W2, b2):
    s = _pool()(X.reshape(B * L).astype(jnp.int32),
                valid_lens.astype(jnp.int32), table)
    out = pl.pallas_call(
        _mlp_body,
        out_shape=jax.ShapeDtypeStruct((B, D), jnp.float32),
    )(s.reshape(B, D), W1, b1.reshape(1, D), W2, b2.reshape(1, D))
    return out
